# Initial kernel scaffold; baseline (speedup 1.0000x reference)
import jax
import jax.numpy as jnp
from jax.experimental import pallas as pl


def kernel(q, kv, attn_sink, topk_idxs):
    raise NotImplementedError("write your pallas kernel here")


# speedup vs baseline: 12.7609x; 12.7609x over previous
import functools

import jax
import jax.numpy as jnp
from jax import lax
from jax.experimental import pallas as pl
from jax.experimental.pallas import tpu as pltpu
from jax.experimental.pallas import tpu_sc as plsc

SOFTMAX_SCALE = 0.08838834764831845
B, SQ, H, D = 8, 8, 16, 128
SKV, K = 8192, 2048
ROWS = B * SQ * K
KV_W = 2 * D

NC, NS, L = 2, 16, 16
NW = NC * NS
RPT = ROWS // NW
CHUNK = 128
NCH = RPT // CHUNK
ROWS_PER_B = SQ * K


def _sc_gather(idx_flat, kv_flat, interpret=False):
    mesh = plsc.VectorSubcoreMesh(core_axis_name="c", subcore_axis_name="s")

    @functools.partial(
        pl.kernel,
        out_type=jax.ShapeDtypeStruct((ROWS, KV_W), jnp.float32),
        mesh=mesh,
        scratch_types=[
            pltpu.VMEM((RPT,), jnp.int32),
            pltpu.VMEM((CHUNK, KV_W), jnp.float32),
            pltpu.VMEM((CHUNK, KV_W), jnp.float32),
            pltpu.SemaphoreType.DMA,
            pltpu.SemaphoreType.DMA,
        ],
        interpret=interpret,
    )
    def gather_kernel(idx_hbm, kv_hbm, out_hbm, idx_v, buf0, buf1, sem0, sem1):
        wid = lax.axis_index("s") * NC + lax.axis_index("c")
        tbase = wid * RPT
        pltpu.sync_copy(idx_hbm.at[pl.ds(tbase, RPT)], idx_v)
        badd = (wid // (ROWS_PER_B // RPT)) * SKV

        def add_body(i, carry):
            off = pl.multiple_of(i * L, L)
            idx_v[pl.ds(off, L)] = idx_v[pl.ds(off, L)] + badd
            return carry

        lax.fori_loop(0, RPT // L, add_body, 0)

        def start_gather(c, buf, sem):
            src = kv_hbm.at[idx_v.at[pl.ds(pl.multiple_of(c * CHUNK, CHUNK), CHUNK)]]
            pltpu.make_async_copy(src, buf, sem).start()

        def wait_gather(c, buf, sem):
            src = kv_hbm.at[idx_v.at[pl.ds(pl.multiple_of(c * CHUNK, CHUNK), CHUNK)]]
            pltpu.make_async_copy(src, buf, sem).wait()

        def writeback(c, buf):
            row = pl.multiple_of(tbase + c * CHUNK, CHUNK)
            pltpu.sync_copy(buf, out_hbm.at[pl.ds(row, CHUNK)])

        start_gather(0, buf0, sem0)

        def loop_body(i, carry):
            a = i * 2
            start_gather(a + 1, buf1, sem1)
            wait_gather(a, buf0, sem0)
            writeback(a, buf0)

            @pl.when(a + 2 < NCH)
            def _():
                start_gather(a + 2, buf0, sem0)

            wait_gather(a + 1, buf1, sem1)
            writeback(a + 1, buf1)
            return carry

        lax.fori_loop(0, NCH // 2, loop_body, 0)

    return gather_kernel(idx_flat, kv_flat)


def _attn_body(q_ref, kv_ref, sink_ref, o_ref):
    q = q_ref[0]
    kv = kv_ref[0]
    k = kv[:, :D]
    v = kv[:, D:]
    logits = lax.dot_general(
        q, k, (((1,), (1,)), ((), ())), preferred_element_type=jnp.float32
    ) * SOFTMAX_SCALE
    sink = sink_ref[...]
    m = jnp.maximum(jnp.max(logits, axis=1, keepdims=True), sink)
    e = jnp.exp(logits - m)
    denom = jnp.sum(e, axis=1, keepdims=True) + jnp.exp(sink - m)
    p = e * (1.0 / denom)
    o_ref[0] = lax.dot_general(
        p, v, (((1,), (0,)), ((), ())), preferred_element_type=jnp.float32
    )


def _tc_attention(q3, kv_sel3, sink_col, interpret=False):
    n = q3.shape[0]
    return pl.pallas_call(
        _attn_body,
        grid=(n,),
        in_specs=[
            pl.BlockSpec((1, H, D), lambda i: (i, 0, 0)),
            pl.BlockSpec((1, K, KV_W), lambda i: (i, 0, 0)),
            pl.BlockSpec((H, 1), lambda i: (0, 0)),
        ],
        out_specs=pl.BlockSpec((1, H, D), lambda i: (i, 0, 0)),
        out_shape=jax.ShapeDtypeStruct((n, H, D), jnp.float32),
        compiler_params=pltpu.CompilerParams(
            dimension_semantics=("arbitrary",),
        ),
        interpret=interpret,
    )(q3, kv_sel3, sink_col)


def kernel(q, kv, attn_sink, topk_idxs):
    idx_flat = topk_idxs.reshape(ROWS)
    kv_flat = kv.reshape(B * SKV, KV_W)
    kv_sel = _sc_gather(idx_flat, kv_flat)
    out = _tc_attention(
        q.reshape(B * SQ, H, D),
        kv_sel.reshape(B * SQ, K, KV_W),
        attn_sink.reshape(H, 1),
    )
    return out.reshape(B, SQ, H, D)


# speedup vs baseline: 13.6808x; 1.0721x over previous
import functools

import jax
import jax.numpy as jnp
from jax import lax
from jax.experimental import pallas as pl
from jax.experimental.pallas import tpu as pltpu
from jax.experimental.pallas import tpu_sc as plsc

SOFTMAX_SCALE = 0.08838834764831845
B, SQ, H, D = 8, 8, 16, 128
SKV, K = 8192, 2048
ROWS = B * SQ * K
KV_W = 2 * D
ROWS_PER_B = SQ * K

NC, NS, L = 2, 16, 16
NW = NC * NS
CHUNK = 128

NSLAB = 4
SLAB_ROWS = ROWS // NSLAB
SLAB_PAIRS = (B * SQ) // NSLAB


def _sc_gather_slab(slab, idx_slab, kv_flat):
    rpt = SLAB_ROWS // NW
    nch = rpt // CHUNK
    mesh = plsc.VectorSubcoreMesh(core_axis_name="c", subcore_axis_name="s")

    @functools.partial(
        pl.kernel,
        out_type=jax.ShapeDtypeStruct((SLAB_ROWS, KV_W), jnp.float32),
        mesh=mesh,
        scratch_types=[
            pltpu.VMEM((rpt,), jnp.int32),
            pltpu.VMEM((CHUNK, KV_W), jnp.float32),
            pltpu.VMEM((CHUNK, KV_W), jnp.float32),
            pltpu.SemaphoreType.DMA,
            pltpu.SemaphoreType.DMA,
        ],
    )
    def gather_kernel(idx_hbm, kv_hbm, out_hbm, idx_v, buf0, buf1, sem0, sem1):
        wid = lax.axis_index("s") * NC + lax.axis_index("c")
        tbase = wid * rpt
        pltpu.sync_copy(idx_hbm.at[pl.ds(tbase, rpt)], idx_v)
        badd = ((slab * SLAB_ROWS + tbase) // ROWS_PER_B) * SKV

        def add_body(i, carry):
            off = pl.multiple_of(i * L, L)
            idx_v[pl.ds(off, L)] = idx_v[pl.ds(off, L)] + badd
            return carry

        lax.fori_loop(0, rpt // L, add_body, 0)

        def start_gather(c, buf, sem):
            src = kv_hbm.at[idx_v.at[pl.ds(pl.multiple_of(c * CHUNK, CHUNK), CHUNK)]]
            pltpu.make_async_copy(src, buf, sem).start()

        def wait_gather(c, buf, sem):
            src = kv_hbm.at[idx_v.at[pl.ds(pl.multiple_of(c * CHUNK, CHUNK), CHUNK)]]
            pltpu.make_async_copy(src, buf, sem).wait()

        def writeback(c, buf):
            row = pl.multiple_of(tbase + c * CHUNK, CHUNK)
            pltpu.sync_copy(buf, out_hbm.at[pl.ds(row, CHUNK)])

        start_gather(0, buf0, sem0)

        def loop_body(i, carry):
            a = i * 2
            start_gather(a + 1, buf1, sem1)
            wait_gather(a, buf0, sem0)
            writeback(a, buf0)

            @pl.when(a + 2 < nch)
            def _():
                start_gather(a + 2, buf0, sem0)

            wait_gather(a + 1, buf1, sem1)
            writeback(a + 1, buf1)
            return carry

        lax.fori_loop(0, nch // 2, loop_body, 0)

    return gather_kernel(idx_slab, kv_flat)


def _attn_body(q_ref, kv_ref, sink_ref, o_ref):
    q = q_ref[0]
    kv = kv_ref[0]
    k = kv[:, :D]
    v = kv[:, D:]
    logits = lax.dot_general(
        q, k, (((1,), (1,)), ((), ())), preferred_element_type=jnp.float32
    ) * SOFTMAX_SCALE
    sink = sink_ref[...]
    m = jnp.maximum(jnp.max(logits, axis=1, keepdims=True), sink)
    e = jnp.exp(logits - m)
    denom = jnp.sum(e, axis=1, keepdims=True) + jnp.exp(sink - m)
    p = e * (1.0 / denom)
    o_ref[0] = lax.dot_general(
        p, v, (((1,), (0,)), ((), ())), preferred_element_type=jnp.float32
    )


def _tc_attention(q3, kv_sel3, sink_col):
    n = q3.shape[0]
    return pl.pallas_call(
        _attn_body,
        grid=(n,),
        in_specs=[
            pl.BlockSpec((1, H, D), lambda i: (i, 0, 0)),
            pl.BlockSpec((1, K, KV_W), lambda i: (i, 0, 0)),
            pl.BlockSpec((H, 1), lambda i: (0, 0)),
        ],
        out_specs=pl.BlockSpec((1, H, D), lambda i: (i, 0, 0)),
        out_shape=jax.ShapeDtypeStruct((n, H, D), jnp.float32),
        compiler_params=pltpu.CompilerParams(
            dimension_semantics=("arbitrary",),
        ),
    )(q3, kv_sel3, sink_col)


def kernel(q, kv, attn_sink, topk_idxs):
    idx_flat = topk_idxs.reshape(NSLAB, SLAB_ROWS)
    kv_flat = kv.reshape(B * SKV, KV_W)
    q4 = q.reshape(NSLAB, SLAB_PAIRS, H, D)
    sink_col = attn_sink.reshape(H, 1)
    outs = []
    for s in range(NSLAB):
        kv_sel = _sc_gather_slab(s, idx_flat[s], kv_flat)
        outs.append(
            _tc_attention(q4[s], kv_sel.reshape(SLAB_PAIRS, K, KV_W), sink_col)
        )
    return jnp.stack(outs).reshape(B, SQ, H, D)


# speedup vs baseline: 13.6831x; 1.0002x over previous
import functools

import jax
import jax.numpy as jnp
from jax import lax
from jax.experimental import pallas as pl
from jax.experimental.pallas import tpu as pltpu
from jax.experimental.pallas import tpu_sc as plsc

SOFTMAX_SCALE = 0.08838834764831845
B, SQ, H, D = 8, 8, 16, 128
SKV, K = 8192, 2048
ROWS = B * SQ * K
ROWS_PER_B = SQ * K

NC, NS, L = 2, 16, 16
NW = NC * NS
CHUNK = 128

NSLAB = 4
SLAB_ROWS = ROWS // NSLAB
SLAB_PAIRS = (B * SQ) // NSLAB


def _sc_gather_slab(slab, idx_slab, kvp):
    rpt = SLAB_ROWS // NW
    nch = rpt // CHUNK
    mesh = plsc.VectorSubcoreMesh(core_axis_name="c", subcore_axis_name="s")

    @functools.partial(
        pl.kernel,
        out_type=jax.ShapeDtypeStruct((SLAB_ROWS, D), jnp.int32),
        mesh=mesh,
        scratch_types=[
            pltpu.VMEM((rpt,), jnp.int32),
            pltpu.VMEM((CHUNK, D), jnp.int32),
            pltpu.VMEM((CHUNK, D), jnp.int32),
            pltpu.SemaphoreType.DMA,
            pltpu.SemaphoreType.DMA,
        ],
    )
    def gather_kernel(idx_hbm, kv_hbm, out_hbm, idx_v, buf0, buf1, sem0, sem1):
        wid = lax.axis_index("s") * NC + lax.axis_index("c")
        tbase = wid * rpt
        pltpu.sync_copy(idx_hbm.at[pl.ds(tbase, rpt)], idx_v)
        badd = ((slab * SLAB_ROWS + tbase) // ROWS_PER_B) * SKV

        def add_body(i, carry):
            off = pl.multiple_of(i * L, L)
            idx_v[pl.ds(off, L)] = idx_v[pl.ds(off, L)] + badd
            return carry

        lax.fori_loop(0, rpt // L, add_body, 0)

        def start_gather(c, buf, sem):
            src = kv_hbm.at[idx_v.at[pl.ds(pl.multiple_of(c * CHUNK, CHUNK), CHUNK)]]
            pltpu.make_async_copy(src, buf, sem).start()

        def wait_gather(c, buf, sem):
            src = kv_hbm.at[idx_v.at[pl.ds(pl.multiple_of(c * CHUNK, CHUNK), CHUNK)]]
            pltpu.make_async_copy(src, buf, sem).wait()

        def writeback(c, buf):
            row = pl.multiple_of(tbase + c * CHUNK, CHUNK)
            pltpu.sync_copy(buf, out_hbm.at[pl.ds(row, CHUNK)])

        start_gather(0, buf0, sem0)

        def loop_body(i, carry):
            a = i * 2
            start_gather(a + 1, buf1, sem1)
            wait_gather(a, buf0, sem0)
            writeback(a, buf0)

            @pl.when(a + 2 < nch)
            def _():
                start_gather(a + 2, buf0, sem0)

            wait_gather(a + 1, buf1, sem1)
            writeback(a + 1, buf1)
            return carry

        lax.fori_loop(0, nch // 2, loop_body, 0)

    return gather_kernel(idx_slab, kvp)


def _attn_body(q_ref, kv_ref, sink_ref, o_ref):
    q = q_ref[0]
    w = kv_ref[0]
    kf = lax.bitcast_convert_type(w << 16, jnp.float32)
    vf = lax.bitcast_convert_type(w & jnp.int32(-65536), jnp.float32)
    kb = kf.astype(jnp.bfloat16)
    vb = vf.astype(jnp.bfloat16)
    logits = lax.dot_general(
        q, kb, (((1,), (1,)), ((), ())), preferred_element_type=jnp.float32
    ) * SOFTMAX_SCALE
    sink = sink_ref[...]
    m = jnp.maximum(jnp.max(logits, axis=1, keepdims=True), sink)
    e = jnp.exp(logits - m)
    denom = jnp.sum(e, axis=1, keepdims=True) + jnp.exp(sink - m)
    p = (e * (1.0 / denom)).astype(jnp.bfloat16)
    o_ref[0] = lax.dot_general(
        p, vb, (((1,), (0,)), ((), ())), preferred_element_type=jnp.float32
    )


def _tc_attention(q3, kv_sel3, sink_col):
    n = q3.shape[0]
    return pl.pallas_call(
        _attn_body,
        grid=(n,),
        in_specs=[
            pl.BlockSpec((1, H, D), lambda i: (i, 0, 0)),
            pl.BlockSpec((1, K, D), lambda i: (i, 0, 0)),
            pl.BlockSpec((H, 1), lambda i: (0, 0)),
        ],
        out_specs=pl.BlockSpec((1, H, D), lambda i: (i, 0, 0)),
        out_shape=jax.ShapeDtypeStruct((n, H, D), jnp.float32),
        compiler_params=pltpu.CompilerParams(
            dimension_semantics=("arbitrary",),
        ),
    )(q3, kv_sel3, sink_col)


def kernel(q, kv, attn_sink, topk_idxs):
    idx_flat = topk_idxs.reshape(NSLAB, SLAB_ROWS)
    kvb = kv.astype(jnp.bfloat16)
    k16 = lax.bitcast_convert_type(kvb[..., :D], jnp.uint16).astype(jnp.uint32)
    v16 = lax.bitcast_convert_type(kvb[..., D:], jnp.uint16).astype(jnp.uint32)
    kvp = lax.bitcast_convert_type(k16 | (v16 << 16), jnp.int32)
    kvp = kvp.reshape(B * SKV, D)
    q4 = q.astype(jnp.bfloat16).reshape(NSLAB, SLAB_PAIRS, H, D)
    sink_col = attn_sink.reshape(H, 1)
    outs = []
    for s in range(NSLAB):
        kv_sel = _sc_gather_slab(s, idx_flat[s], kvp)
        outs.append(
            _tc_attention(q4[s], kv_sel.reshape(SLAB_PAIRS, K, D), sink_col)
        )
    return jnp.stack(outs).reshape(B, SQ, H, D)


# speedup vs baseline: 13.7736x; 1.0066x over previous
import functools

import jax
import jax.numpy as jnp
from jax import lax
from jax.experimental import pallas as pl
from jax.experimental.pallas import tpu as pltpu
from jax.experimental.pallas import tpu_sc as plsc

SOFTMAX_SCALE = 0.08838834764831845
B, SQ, H, D = 8, 8, 16, 128
SKV, K = 8192, 2048
ROWS = B * SQ * K
ROWS_PER_B = SQ * K

NC, NS, L = 2, 16, 16
NW = NC * NS
CHUNK = 128

NSLAB = 4
SLAB_ROWS = ROWS // NSLAB
SLAB_PAIRS = (B * SQ) // NSLAB


def _sc_gather_slab(slab, idx_slab, kvp):
    rpt = SLAB_ROWS // NW
    nch = rpt // CHUNK
    mesh = plsc.VectorSubcoreMesh(core_axis_name="c", subcore_axis_name="s")

    @functools.partial(
        pl.kernel,
        out_type=jax.ShapeDtypeStruct((SLAB_ROWS, D), jnp.int32),
        mesh=mesh,
        scratch_types=[
            pltpu.VMEM((rpt,), jnp.int32),
            pltpu.VMEM((CHUNK, D), jnp.int32),
            pltpu.VMEM((CHUNK, D), jnp.int32),
            pltpu.SemaphoreType.DMA,
            pltpu.SemaphoreType.DMA,
        ],
    )
    def gather_kernel(idx_hbm, kv_hbm, out_hbm, idx_v, buf0, buf1, sem0, sem1):
        wid = lax.axis_index("s") * NC + lax.axis_index("c")
        tbase = wid * rpt
        pltpu.sync_copy(idx_hbm.at[pl.ds(tbase, rpt)], idx_v)
        badd = ((slab * SLAB_ROWS + tbase) // ROWS_PER_B) * SKV

        def add_body(i, carry):
            off = pl.multiple_of(i * L, L)
            idx_v[pl.ds(off, L)] = idx_v[pl.ds(off, L)] + badd
            return carry

        lax.fori_loop(0, rpt // L, add_body, 0)

        def start_gather(c, buf, sem):
            src = kv_hbm.at[idx_v.at[pl.ds(pl.multiple_of(c * CHUNK, CHUNK), CHUNK)]]
            pltpu.make_async_copy(src, buf, sem).start()

        def wait_gather(c, buf, sem):
            src = kv_hbm.at[idx_v.at[pl.ds(pl.multiple_of(c * CHUNK, CHUNK), CHUNK)]]
            pltpu.make_async_copy(src, buf, sem).wait()

        def writeback(c, buf):
            row = pl.multiple_of(tbase + c * CHUNK, CHUNK)
            pltpu.sync_copy(buf, out_hbm.at[pl.ds(row, CHUNK)])

        start_gather(0, buf0, sem0)

        def loop_body(i, carry):
            a = i * 2
            start_gather(a + 1, buf1, sem1)
            wait_gather(a, buf0, sem0)
            writeback(a, buf0)

            @pl.when(a + 2 < nch)
            def _():
                start_gather(a + 2, buf0, sem0)

            wait_gather(a + 1, buf1, sem1)
            writeback(a + 1, buf1)
            return carry

        lax.fori_loop(0, nch // 2, loop_body, 0)

    return gather_kernel(idx_slab, kvp)


def _attn_body(q_ref, kv_ref, sink_ref, o_ref):
    q = q_ref[0]
    w = kv_ref[0]
    kf = lax.bitcast_convert_type(w << 16, jnp.float32)
    vf = lax.bitcast_convert_type(w & jnp.int32(-65536), jnp.float32)
    kb = kf.astype(jnp.bfloat16)
    vb = vf.astype(jnp.bfloat16)
    logits = lax.dot_general(
        q, kb, (((1,), (1,)), ((), ())), preferred_element_type=jnp.float32
    ) * SOFTMAX_SCALE
    sink = sink_ref[...]
    m = jnp.maximum(jnp.max(logits, axis=1, keepdims=True), sink)
    e = jnp.exp(logits - m)
    denom = jnp.sum(e, axis=1, keepdims=True) + jnp.exp(sink - m)
    p = (e * (1.0 / denom)).astype(jnp.bfloat16)
    o_ref[0] = lax.dot_general(
        p, vb, (((1,), (0,)), ((), ())), preferred_element_type=jnp.float32
    )


def _tc_attention(q3, kv_sel3, sink_col):
    n = q3.shape[0]
    return pl.pallas_call(
        _attn_body,
        grid=(n,),
        in_specs=[
            pl.BlockSpec((1, H, D), lambda i: (i, 0, 0)),
            pl.BlockSpec((1, K, D), lambda i: (i, 0, 0)),
            pl.BlockSpec((H, 1), lambda i: (0, 0)),
        ],
        out_specs=pl.BlockSpec((1, H, D), lambda i: (i, 0, 0)),
        out_shape=jax.ShapeDtypeStruct((n, H, D), jnp.float32),
        compiler_params=pltpu.CompilerParams(
            dimension_semantics=("arbitrary",),
        ),
    )(q3, kv_sel3, sink_col)


def _pack_kv(kv):
    kvb = kv.astype(jnp.bfloat16)
    k16 = lax.bitcast_convert_type(kvb[..., :D], jnp.uint16).astype(jnp.uint32)
    v16 = lax.bitcast_convert_type(kvb[..., D:], jnp.uint16).astype(jnp.uint32)
    kvp = lax.bitcast_convert_type(k16 | (v16 << 16), jnp.int32)
    return kvp.reshape(-1, D)


BATCHES_PER_SLAB = B // NSLAB


def kernel(q, kv, attn_sink, topk_idxs):
    idx_flat = topk_idxs.reshape(NSLAB, SLAB_ROWS)
    kvps = [
        _pack_kv(kv[s * BATCHES_PER_SLAB:(s + 1) * BATCHES_PER_SLAB])
        for s in range(NSLAB)
    ]
    q4 = q.astype(jnp.bfloat16).reshape(NSLAB, SLAB_PAIRS, H, D)
    sink_col = attn_sink.reshape(H, 1)
    kv_sels = [_sc_gather_slab(0, idx_flat[0], kvps[0])]
    outs = []
    for s in range(NSLAB):
        if s + 1 < NSLAB:
            kv_sels.append(_sc_gather_slab(0, idx_flat[s + 1], kvps[s + 1]))
        outs.append(
            _tc_attention(q4[s], kv_sels[s].reshape(SLAB_PAIRS, K, D), sink_col)
        )
    return jnp.stack(outs).reshape(B, SQ, H, D)
